# Initial kernel scaffold; baseline (speedup 1.0000x reference)
import jax
import jax.numpy as jnp
from jax.experimental import pallas as pl


def kernel(I, flow):
    raise NotImplementedError("write your pallas kernel here")


# speedup vs baseline: 1.4055x; 1.4055x over previous
import functools

import jax
import jax.numpy as jnp
from jax import lax
from jax.experimental import pallas as pl
from jax.experimental.pallas import tpu as pltpu
from jax.experimental.pallas import tpu_sc as plsc

B, C, D, H, W = 2, 1, 128, 192, 192
HW = H * W
DHW = D * HW
N = B * DHW

NC, NS, LANES = 2, 16, 16
NW = NC * NS
NPW = N // NW
WPB = NW // B
K = 3072
ROWS_PER_CHUNK = K // W
GROUPS_PER_ROW = W // LANES
CHUNKS = NPW // K


def _floor_i32(v):
    ti = v.astype(jnp.int32)
    tf = ti.astype(jnp.float32)
    return jnp.where(tf > v, ti - 1, ti)


def _warp_body(I_hbm, flow_hbm, out_hbm, dxv, dyv, dzv,
               ia, ib, ic, id_, ie, if_, ig, ih,
               va, vb, vc, vd, ve, vf, vg, vh, outv, sem):
    idxs = (ia, ib, ic, id_, ie, if_, ig, ih)
    vals = (va, vb, vc, vd, ve, vf, vg, vh)
    cid = lax.axis_index("c")
    sid = lax.axis_index("s")
    wid = sid * NC + cid
    b = wid // WPB
    widx = wid % WPB
    o_batch0 = widx * NPW
    bbase = b * DHW
    fbase = b * 3 * DHW

    def chunk_body(g, carry):
        o = o_batch0 + g * K
        pltpu.sync_copy(flow_hbm.at[pl.ds(fbase + o, K)], dxv)
        pltpu.sync_copy(flow_hbm.at[pl.ds(fbase + DHW + o, K)], dyv)
        pltpu.sync_copy(flow_hbm.at[pl.ds(fbase + 2 * DHW + o, K)], dzv)
        row0 = o // W

        def row_body(t, c2):
            r = row0 + t
            z = r // H
            y = r - z * H
            yf = y.astype(jnp.float32)
            zf = z.astype(jnp.float32)
            for j in range(GROUPS_PER_ROW):
                off = t * W + j * LANES
                sl = pl.ds(off, LANES)
                lane = lax.iota(jnp.int32, LANES).astype(jnp.float32) + float(j * LANES)
                xs = dxv[sl] + lane
                ys = dyv[sl] + yf
                zs = dzv[sl] + zf
                x0 = _floor_i32(xs)
                y0 = _floor_i32(ys)
                z0 = _floor_i32(zs)
                x0c = jnp.clip(x0, 0, W - 1)
                x1c = jnp.clip(x0 + 1, 0, W - 1)
                y0c = jnp.clip(y0, 0, H - 1)
                y1c = jnp.clip(y0 + 1, 0, H - 1)
                z0c = jnp.clip(z0, 0, D - 1)
                z1c = jnp.clip(z0 + 1, 0, D - 1)
                dxv[sl] = x1c.astype(jnp.float32) - xs
                dyv[sl] = y1c.astype(jnp.float32) - ys
                dzv[sl] = z1c.astype(jnp.float32) - zs
                y0w = y0c * W
                y1w = y1c * W
                zb0 = z0c * HW + bbase
                zb1 = z1c * HW + bbase
                a00 = x0c + y0w
                a01 = x0c + y1w
                a10 = x1c + y0w
                a11 = x1c + y1w
                ia[sl] = a00 + zb0
                ib[sl] = a01 + zb0
                ic[sl] = a10 + zb0
                id_[sl] = a11 + zb0
                ie[sl] = a00 + zb1
                if_[sl] = a01 + zb1
                ig[sl] = a10 + zb1
                ih[sl] = a11 + zb1
            return c2

        lax.fori_loop(0, ROWS_PER_CHUNK, row_body, 0)

        copies = [
            pltpu.async_copy(I_hbm.at[idx_ref], val_ref, sem)
            for idx_ref, val_ref in zip(idxs, vals)
        ]
        for cp in copies:
            cp.wait()

        def comb_body(i, c2):
            sl = pl.ds(i * LANES, LANES)
            fx = dxv[sl]
            fy = dyv[sl]
            fz = dzv[sl]
            ex = 1.0 - fx
            ey = 1.0 - fy
            ez = 1.0 - fz
            t00 = va[sl] * fy + vb[sl] * ey
            t01 = vc[sl] * fy + vd[sl] * ey
            t10 = ve[sl] * fy + vf[sl] * ey
            t11 = vg[sl] * fy + vh[sl] * ey
            u0 = t00 * fx + t01 * ex
            u1 = t10 * fx + t11 * ex
            outv[sl] = u0 * fz + u1 * ez
            return c2

        lax.fori_loop(0, K // LANES, comb_body, 0)

        pltpu.sync_copy(outv, out_hbm.at[pl.ds(bbase + o, K)])
        return carry

    lax.fori_loop(0, CHUNKS, chunk_body, 0)


@jax.jit
def _warp(I_flat, flow_flat):
    mesh = plsc.VectorSubcoreMesh(core_axis_name="c", subcore_axis_name="s")
    f = functools.partial(
        pl.kernel,
        mesh=mesh,
        out_type=jax.ShapeDtypeStruct((N,), jnp.float32),
        scratch_types=[
            pltpu.VMEM((K,), jnp.float32),
            pltpu.VMEM((K,), jnp.float32),
            pltpu.VMEM((K,), jnp.float32),
        ] + [pltpu.VMEM((K,), jnp.int32) for _ in range(8)]
          + [pltpu.VMEM((K,), jnp.float32) for _ in range(8)]
          + [
            pltpu.VMEM((K,), jnp.float32),
            pltpu.SemaphoreType.DMA,
        ],
    )(_warp_body)
    return f(I_flat, flow_flat)


def kernel(I, flow):
    out = _warp(I.reshape(-1), flow.reshape(-1))
    return out.reshape(B, C, D, H, W)
